# Initial kernel scaffold; baseline (speedup 1.0000x reference)
import functools

import jax
import jax.numpy as jnp
from jax import lax
from jax.experimental import pallas as pl
from jax.experimental.pallas import tpu as pltpu
from jax.experimental.pallas import tpu_sc as plsc

_N0, _N1, _N2 = 10000, 4000, 1000
_E1, _E2 = 320000, 64000
_IN_F, _H_F, _N_CLS = 128, 256, 64

_NC, _NS = 2, 16
_NW = _NC * _NS
_K = 128


def _cdiv(a, b):
    return (a + b - 1) // b


def _make_edge_agg(width, nchunks, acc_rows):
    rows_per_tile = acc_rows // _NS
    mesh = plsc.VectorSubcoreMesh(core_axis_name="c", subcore_axis_name="s")

    @functools.partial(
        pl.kernel,
        out_type=jax.ShapeDtypeStruct((_NC, acc_rows, width), jnp.float32),
        mesh=mesh,
        scratch_types=[
            pltpu.VMEM((nchunks, _K), jnp.int32),
            pltpu.VMEM((nchunks, _K), jnp.int32),
            pltpu.VMEM((_K, width), jnp.float32),
            pltpu.VMEM_SHARED((acc_rows, width), jnp.float32),
            pltpu.SemaphoreType.DMA,
        ],
    )
    def agg(table_hbm, src_hbm, dst_hbm, out_hbm, idxs_v, idxd_v, rows_v,
            acc_sh, sem):
        cid = lax.axis_index("c")
        sid = lax.axis_index("s")
        wid = sid * _NC + cid

        def _zrow(r, _):
            def _zcol(c, _):
                rows_v[r, pl.ds(c * 16, 16)] = jnp.zeros((16,), jnp.float32)
                return ()
            return lax.fori_loop(0, width // 16, _zcol, ())
        lax.fori_loop(0, _K, _zrow, ())
        base = sid * rows_per_tile
        def _zacc(i, _):
            pltpu.sync_copy(rows_v, acc_sh.at[pl.ds(base + i * _K, _K)])
            return ()
        lax.fori_loop(0, rows_per_tile // _K, _zacc, ())
        if rows_per_tile % _K:
            pltpu.sync_copy(
                rows_v.at[pl.ds(0, rows_per_tile % _K)],
                acc_sh.at[pl.ds(base + (rows_per_tile // _K) * _K,
                                rows_per_tile % _K)])
        plsc.subcore_barrier()

        pltpu.sync_copy(src_hbm.at[wid], idxs_v)
        pltpu.sync_copy(dst_hbm.at[wid], idxd_v)

        def body(j, _):
            pltpu.async_copy(table_hbm.at[idxs_v.at[j]], rows_v, sem).wait()
            pltpu.sync_copy(rows_v, acc_sh.at[idxd_v.at[j]], add=True)
            return ()
        lax.fori_loop(0, nchunks, body, ())

        plsc.subcore_barrier()
        pltpu.sync_copy(acc_sh.at[pl.ds(base, rows_per_tile)],
                        out_hbm.at[cid, pl.ds(base, rows_per_tile)])

    return agg


def _pad_edges(src, dst, nchunks, junk_dst):
    e = src.shape[0]
    pad = _NW * nchunks * _K - e
    src = jnp.concatenate([src, jnp.zeros((pad,), jnp.int32)])
    dst = jnp.concatenate([dst, jnp.full((pad,), junk_dst, jnp.int32)])
    return src.reshape(_NW, nchunks, _K), dst.reshape(_NW, nchunks, _K)


_C1 = _cdiv(_E1, _NW * _K)
_C2 = _cdiv(_E2, _NW * _K)
_G1 = _IN_F + 16
_G2 = _N_CLS + 16
_ACC1 = 4096
_ACC2 = 1024

_agg1 = _make_edge_agg(_G1, _C1, _ACC1)
_agg2 = _make_edge_agg(_G2, _C2, _ACC2)


def _tc1_body(x4_ref, parts_ref, w1s_ref, w1n_ref, b1_ref, w2n_ref,
              h_ref, hwe_ref):
    acc = parts_ref[0] + parts_ref[1]
    deg = jnp.maximum(acc[:, _IN_F:_IN_F + 1], 1.0)
    hn = acc[:, :_IN_F] / deg
    h = x4_ref[...] @ w1s_ref[...] + hn @ w1n_ref[...] + b1_ref[...]
    h = jnp.maximum(h, 0.0)
    h_ref[...] = h
    onehot = jnp.where(
        lax.broadcasted_iota(jnp.int32, (1, _G2), 1) == _N_CLS, 1.0, 0.0)
    hwe_ref[...] = h @ w2n_ref[...] + onehot


def _tc2_body(h_ref, parts_ref, w2s_ref, b2_ref, out_ref):
    acc = parts_ref[0] + parts_ref[1]
    deg = jnp.maximum(acc[:_N2, _N_CLS:_N_CLS + 1], 1.0)
    hn = acc[:_N2, :_N_CLS] / deg
    out_ref[...] = h_ref[...] @ w2s_ref[...] + hn + b2_ref[...]


_BLK1 = 500


def kernel(x, edge_index1, edge_index2, W1_self, W1_neigh, b1,
           W2_self, W2_neigh, b2):
    xe = jnp.concatenate(
        [x, jnp.ones((_N0, 1), jnp.float32), jnp.zeros((_N0, 15), jnp.float32)],
        axis=1)
    s1, d1 = _pad_edges(edge_index1[0], edge_index1[1], _C1, _N1)
    parts1 = _agg1(xe, s1, d1)

    w2n_pad = jnp.pad(W2_neigh, ((0, 0), (0, _G2 - _N_CLS)))
    h, hwe = pl.pallas_call(
        _tc1_body,
        grid=(_N1 // _BLK1,),
        in_specs=[
            pl.BlockSpec((_BLK1, _IN_F), lambda i: (i, 0)),
            pl.BlockSpec((_NC, _BLK1, _G1), lambda i: (0, i, 0)),
            pl.BlockSpec((_IN_F, _H_F), lambda i: (0, 0)),
            pl.BlockSpec((_IN_F, _H_F), lambda i: (0, 0)),
            pl.BlockSpec((1, _H_F), lambda i: (0, 0)),
            pl.BlockSpec((_H_F, _G2), lambda i: (0, 0)),
        ],
        out_specs=[
            pl.BlockSpec((_BLK1, _H_F), lambda i: (i, 0)),
            pl.BlockSpec((_BLK1, _G2), lambda i: (i, 0)),
        ],
        out_shape=[
            jax.ShapeDtypeStruct((_N1, _H_F), jnp.float32),
            jax.ShapeDtypeStruct((_N1, _G2), jnp.float32),
        ],
    )(x[:_N1], parts1[:, :_N1, :], W1_self, W1_neigh, b1.reshape(1, _H_F),
      w2n_pad)

    s2, d2 = _pad_edges(edge_index2[0], edge_index2[1], _C2, _N2)
    parts2 = _agg2(hwe, s2, d2)

    out = pl.pallas_call(
        _tc2_body,
        out_shape=jax.ShapeDtypeStruct((_N2, _N_CLS), jnp.float32),
    )(h[:_N2], parts2, W2_self, b2.reshape(1, _N_CLS))
    return out


# speedup vs baseline: 5.0884x; 5.0884x over previous
import functools

import jax
import jax.numpy as jnp
from jax import lax
from jax.experimental import pallas as pl
from jax.experimental.pallas import tpu as pltpu
from jax.experimental.pallas import tpu_sc as plsc

_N0, _N1, _N2 = 10000, 4000, 1000
_E1, _E2 = 320000, 64000
_IN_F, _H_F, _N_CLS = 128, 256, 64

_NC, _NS = 2, 16
_NW = _NC * _NS
_K = 128


def _cdiv(a, b):
    return (a + b - 1) // b


def _make_edge_agg(width, nchunks, acc_rows):
    rows_per_tile = acc_rows // _NS
    mesh = plsc.VectorSubcoreMesh(core_axis_name="c", subcore_axis_name="s")

    @functools.partial(
        pl.kernel,
        out_type=jax.ShapeDtypeStruct((_NC, acc_rows, width), jnp.float32),
        mesh=mesh,
        scratch_types=[
            pltpu.VMEM((nchunks, _K), jnp.int32),
            pltpu.VMEM((nchunks, _K), jnp.int32),
            pltpu.VMEM((_K, width), jnp.float32),
            pltpu.VMEM_SHARED((acc_rows, width), jnp.float32),
            pltpu.SemaphoreType.DMA,
        ],
        compiler_params=pltpu.CompilerParams(use_tc_tiling_on_sc=False),
    )
    def agg(table_hbm, src_hbm, dst_hbm, out_hbm, idxs_v, idxd_v, rows_v,
            acc_sh, sem):
        cid = lax.axis_index("c")
        sid = lax.axis_index("s")
        wid = sid * _NC + cid

        def _zrow(r, _):
            def _zcol(c, _):
                rows_v[r, pl.ds(c * 16, 16)] = jnp.zeros((16,), jnp.float32)
                return ()
            return lax.fori_loop(0, width // 16, _zcol, ())
        lax.fori_loop(0, _K, _zrow, ())
        base = sid * rows_per_tile
        def _zacc(i, _):
            pltpu.sync_copy(rows_v, acc_sh.at[pl.ds(base + i * _K, _K)])
            return ()
        lax.fori_loop(0, rows_per_tile // _K, _zacc, ())
        if rows_per_tile % _K:
            pltpu.sync_copy(
                rows_v.at[pl.ds(0, rows_per_tile % _K)],
                acc_sh.at[pl.ds(base + (rows_per_tile // _K) * _K,
                                rows_per_tile % _K)])
        plsc.subcore_barrier()

        pltpu.sync_copy(src_hbm.at[wid], idxs_v)
        pltpu.sync_copy(dst_hbm.at[wid], idxd_v)

        def body(j, _):
            pltpu.async_copy(table_hbm.at[idxs_v.at[j]], rows_v, sem).wait()
            pltpu.sync_copy(rows_v, acc_sh.at[idxd_v.at[j]], add=True)
            return ()
        lax.fori_loop(0, nchunks, body, ())

        plsc.subcore_barrier()
        pltpu.sync_copy(acc_sh.at[pl.ds(base, rows_per_tile)],
                        out_hbm.at[cid, pl.ds(base, rows_per_tile)])

    return agg


def _pad_edges(src, dst, nchunks, junk_dst):
    e = src.shape[0]
    pad = _NW * nchunks * _K - e
    src = jnp.concatenate([src, jnp.zeros((pad,), jnp.int32)])
    dst = jnp.concatenate([dst, jnp.full((pad,), junk_dst, jnp.int32)])
    return src.reshape(_NW, nchunks, _K), dst.reshape(_NW, nchunks, _K)


_C1 = _cdiv(_E1, _NW * _K)
_C2 = _cdiv(_E2, _NW * _K)
_G1 = _IN_F + 16
_G2 = _N_CLS + 16
_ACC1 = 4096
_ACC2 = 1024

_agg1 = _make_edge_agg(_G1, _C1, _ACC1)
_agg2 = _make_edge_agg(_G2, _C2, _ACC2)


def _tc1_body(x4_ref, parts_ref, w1s_ref, w1n_ref, b1_ref, w2n_ref,
              h_ref, hwe_ref):
    acc = parts_ref[0] + parts_ref[1]
    deg = jnp.maximum(acc[:, _IN_F:_IN_F + 1], 1.0)
    hn = acc[:, :_IN_F] / deg
    h = x4_ref[...] @ w1s_ref[...] + hn @ w1n_ref[...] + b1_ref[...]
    h = jnp.maximum(h, 0.0)
    h_ref[...] = h
    onehot = jnp.where(
        lax.broadcasted_iota(jnp.int32, (1, _G2), 1) == _N_CLS, 1.0, 0.0)
    hwe_ref[...] = h @ w2n_ref[...] + onehot


def _tc2_body(h_ref, parts_ref, w2s_ref, b2_ref, out_ref):
    acc = parts_ref[0] + parts_ref[1]
    deg = jnp.maximum(acc[:_N2, _N_CLS:_N_CLS + 1], 1.0)
    hn = acc[:_N2, :_N_CLS] / deg
    out_ref[...] = h_ref[...] @ w2s_ref[...] + hn + b2_ref[...]


_BLK1 = 400


def kernel(x, edge_index1, edge_index2, W1_self, W1_neigh, b1,
           W2_self, W2_neigh, b2):
    xe = jnp.concatenate(
        [x, jnp.ones((_N0, 1), jnp.float32), jnp.zeros((_N0, 15), jnp.float32)],
        axis=1)
    s1, d1 = _pad_edges(edge_index1[0], edge_index1[1], _C1, _N1)
    parts1 = _agg1(xe, s1, d1)

    w2n_pad = jnp.pad(W2_neigh, ((0, 0), (0, _G2 - _N_CLS)))
    h, hwe = pl.pallas_call(
        _tc1_body,
        grid=(_N1 // _BLK1,),
        in_specs=[
            pl.BlockSpec((_BLK1, _IN_F), lambda i: (i, 0)),
            pl.BlockSpec((_NC, _BLK1, _G1), lambda i: (0, i, 0)),
            pl.BlockSpec((_IN_F, _H_F), lambda i: (0, 0)),
            pl.BlockSpec((_IN_F, _H_F), lambda i: (0, 0)),
            pl.BlockSpec((1, _H_F), lambda i: (0, 0)),
            pl.BlockSpec((_H_F, _G2), lambda i: (0, 0)),
        ],
        out_specs=[
            pl.BlockSpec((_BLK1, _H_F), lambda i: (i, 0)),
            pl.BlockSpec((_BLK1, _G2), lambda i: (i, 0)),
        ],
        out_shape=[
            jax.ShapeDtypeStruct((_N1, _H_F), jnp.float32),
            jax.ShapeDtypeStruct((_N1, _G2), jnp.float32),
        ],
    )(x[:_N1], parts1[:, :_N1, :], W1_self, W1_neigh, b1.reshape(1, _H_F),
      w2n_pad)

    s2, d2 = _pad_edges(edge_index2[0], edge_index2[1], _C2, _N2)
    parts2 = _agg2(hwe, s2, d2)

    out = pl.pallas_call(
        _tc2_body,
        out_shape=jax.ShapeDtypeStruct((_N2, _N_CLS), jnp.float32),
    )(h[:_N2], parts2, W2_self, b2.reshape(1, _N_CLS))
    return out
